# baseline (speedup 1.0000x reference)
import functools

import jax
import jax.numpy as jnp
from jax import lax
from jax.experimental import pallas as pl
from jax.experimental.pallas import tpu as pltpu
from jax.experimental.pallas import tpu_sc as plsc

N = 10000
E = 160000
H = 256
OUT = 128
NUM_NET = 4

NC = 2
NS = 16
NW = NC * NS
NPAD = 10240
NR = NPAD // NW
NRT = NR + 8
CE = 64
SE = 2048
NCI = SE // CE
EP = E + SE + CE
ROWS = 512
NEG = -3.0e38

_mesh = plsc.VectorSubcoreMesh(
    core_axis_name="c", subcore_axis_name="s", num_cores=NC, num_subcores=NS)


def _agg_body(y_h, srcs_h, dsts_h, meta_h, out_h,
              src_blk, dst_blk, rows2, acc_b, meta_v, sem0, sem1):
    t = lax.axis_index("s") * NC + lax.axis_index("c")
    nr0 = t * NR
    zero16 = jnp.zeros((16,), jnp.float32)
    pltpu.sync_copy(meta_h, meta_v)
    mv = meta_v[pl.ds(t * 16, 16)]
    elo = mv[0]
    ehi = mv[1]

    def zrow(i, carry):
        for c in range(H // 16):
            acc_b[i, pl.ds(c * 16, 16)] = zero16
        return carry

    lax.fori_loop(0, NRT, zrow, 0)
    sems = (sem0, sem1)
    nsc = lax.div(ehi - elo + (SE - 1), jnp.int32(SE))

    def sc_loop(si, carry):
        sbase = pl.multiple_of(elo + si * SE, 8)
        pltpu.sync_copy(srcs_h.at[pl.ds(sbase, SE)], src_blk)
        pltpu.sync_copy(dsts_h.at[pl.ds(sbase, SE)], dst_blk)
        nin = jnp.minimum(jnp.int32(NCI),
                          lax.div(ehi - sbase + (CE - 1), jnp.int32(CE)))
        pltpu.async_copy(y_h.at[src_blk.at[pl.ds(0, CE)]], rows2.at[0], sem0)
        npair = lax.div(nin + 1, jnp.int32(2))

        def pair(j2, c2):
            for b in (0, 1):
                j = j2 * 2 + b

                @pl.when(j < nin)
                def _():
                    pltpu.make_async_copy(
                        y_h.at[src_blk.at[pl.ds(0, CE)]], rows2.at[b],
                        sems[b]).wait()

                    @pl.when(j + 1 < nin)
                    def _():
                        pltpu.async_copy(
                            y_h.at[src_blk.at[pl.ds((j + 1) * CE, CE)]],
                            rows2.at[1 - b], sems[1 - b])

                    def grp(g, c3):
                        dvr = dst_blk[pl.ds(j * CE + g * 16, 16)] - nr0
                        ok = (dvr >= 0) & (dvr < NR)
                        dv = jnp.where(ok, dvr, jnp.int32(NR))
                        for jj in range(16):
                            row = dv[jj]
                            e = g * 16 + jj
                            for c in range(H // 16):
                                plsc.addupdate(
                                    acc_b.at[row, pl.ds(c * 16, 16)],
                                    rows2[b, e, pl.ds(c * 16, 16)])
                        return c3

                    lax.fori_loop(0, CE // 16, grp, 0)
            return c2

        lax.fori_loop(0, npair, pair, 0)
        return carry

    lax.fori_loop(0, nsc, sc_loop, 0)
    pltpu.sync_copy(acc_b.at[pl.ds(0, NR)], out_h.at[pl.ds(nr0, NR)])


_agg_call = pl.kernel(
    _agg_body,
    out_type=jax.ShapeDtypeStruct((NPAD, H), jnp.float32),
    mesh=_mesh,
    scratch_types=[
        pltpu.VMEM((SE,), jnp.int32),
        pltpu.VMEM((SE,), jnp.int32),
        pltpu.VMEM((2, CE, H), jnp.float32),
        pltpu.VMEM((NRT, H), jnp.float32),
        pltpu.VMEM((NW * 16,), jnp.int32),
        pltpu.SemaphoreType.DMA,
        pltpu.SemaphoreType.DMA,
    ],
)


def _deg_body(idxs_h, meta_h, out_h, idx_blk, acc_b, meta_v):
    t = lax.axis_index("s") * NC + lax.axis_index("c")
    nr0 = t * NR
    zero16 = jnp.zeros((16,), jnp.float32)
    pltpu.sync_copy(meta_h, meta_v)
    onehot = jnp.where(lax.iota(jnp.int32, 16) == 0, 1.0, 0.0)

    for p in range(4):
        mv = meta_v[pl.ds((p * NW + t) * 16, 16)]
        elo = mv[0]
        ehi = mv[1]

        def zrow(i, carry):
            acc_b[i, pl.ds(0, 16)] = zero16
            return carry

        lax.fori_loop(0, NRT, zrow, 0)
        nsc = lax.div(ehi - elo + (SE - 1), jnp.int32(SE))

        def sc_loop(si, carry):
            sbase = pl.multiple_of(elo + si * SE, 8)
            pltpu.sync_copy(idxs_h.at[pl.ds(sbase, SE)], idx_blk)
            ngr = jnp.minimum(jnp.int32(SE // 16),
                              lax.div(ehi - sbase + 15, jnp.int32(16)))

            def grp(g, c2):
                dvr = idx_blk[pl.ds(g * 16, 16)] - nr0
                ok = (dvr >= 0) & (dvr < NR)
                dv = jnp.where(ok, dvr, jnp.int32(NR))
                for jj in range(16):
                    row = dv[jj]
                    plsc.addupdate(acc_b.at[row, pl.ds(0, 16)], onehot)
                return c2

            lax.fori_loop(0, ngr, grp, 0)
            return carry

        lax.fori_loop(0, nsc, sc_loop, 0)
        pltpu.sync_copy(acc_b.at[pl.ds(0, NR)],
                        out_h.at[pl.ds(p * NPAD + nr0, NR)])


_deg_call = pl.kernel(
    _deg_body,
    out_type=jax.ShapeDtypeStruct((4 * NPAD, 16), jnp.float32),
    mesh=_mesh,
    scratch_types=[
        pltpu.VMEM((SE,), jnp.int32),
        pltpu.VMEM((NRT, 16), jnp.float32),
        pltpu.VMEM((4 * NW * 16,), jnp.int32),
    ],
)


def _scale_body(x_ref, dego_ref, o_ref):
    dego = dego_ref[...]
    ns = jnp.where(dego > 0, lax.rsqrt(dego), 0.0)
    o_ref[...] = x_ref[...] * ns


def _dense_mid_body(agg_ref, dego_ref, degi_ref, w_ref, b_ref, o_ref):
    dego = dego_ref[...]
    degi = degi_ref[...]
    ns = jnp.where(dego > 0, lax.rsqrt(dego), 0.0)
    nd = jnp.where(degi > 0, lax.rsqrt(degi), 0.0)
    acc = jnp.dot(agg_ref[...], w_ref[...], preferred_element_type=jnp.float32)
    o_ref[...] = acc * (ns * nd) + ns * b_ref[...]


def _dense_last_body(agg_ref, dego_ref, degi_ref, w_ref, b_ref, o_ref):
    degi = degi_ref[...]
    nd = jnp.where(degi > 0, lax.rsqrt(degi), 0.0)
    acc = jnp.dot(agg_ref[...], w_ref[...], preferred_element_type=jnp.float32)
    o_ref[...] = acc * nd + b_ref[...]


def _row_call(body, n_in):
    grid = (NPAD // ROWS,)
    specs = [pl.BlockSpec((ROWS, H), lambda i: (i, 0)),
             pl.BlockSpec((ROWS, 1), lambda i: (i, 0)),
             pl.BlockSpec((ROWS, 1), lambda i: (i, 0)),
             pl.BlockSpec((H, H), lambda i: (0, 0)),
             pl.BlockSpec((1, H), lambda i: (0, 0))][:n_in]
    return pl.pallas_call(
        body, grid=grid, in_specs=specs,
        out_specs=pl.BlockSpec((ROWS, H), lambda i: (i, 0)),
        out_shape=jax.ShapeDtypeStruct((NPAD, H), jnp.float32))


def _pool_body(h1_ref, h0_ref, o1_ref, o0_ref):
    i = pl.program_id(0)
    rows = lax.broadcasted_iota(jnp.int32, (ROWS, 1), 0) + i * ROWS
    valid = rows < N
    m1 = jnp.max(jnp.where(valid, h1_ref[...], NEG), axis=0, keepdims=True)
    m0 = jnp.max(jnp.where(valid, h0_ref[...], NEG), axis=0, keepdims=True)

    @pl.when(i == 0)
    def _():
        o1_ref[...] = jnp.full((1, H), NEG, jnp.float32)
        o0_ref[...] = jnp.full((1, H), NEG, jnp.float32)

    o1_ref[...] = jnp.maximum(o1_ref[...], m1)
    o0_ref[...] = jnp.maximum(o0_ref[...], m0)


_pool_call = pl.pallas_call(
    _pool_body, grid=(NPAD // ROWS,),
    in_specs=[pl.BlockSpec((ROWS, H), lambda i: (i, 0)),
              pl.BlockSpec((ROWS, H), lambda i: (i, 0))],
    out_specs=[pl.BlockSpec((1, H), lambda i: (0, 0)),
               pl.BlockSpec((1, H), lambda i: (0, 0))],
    out_shape=[jax.ShapeDtypeStruct((1, H), jnp.float32),
               jax.ShapeDtypeStruct((1, H), jnp.float32)])


def _mlp_body(m1_ref, m0_ref, wf_ref, bf_ref, wo_ref, bo_ref, o_ref):
    x11 = (m1_ref[...] + m0_ref[...]) * (1.0 / NUM_NET)
    xc = jax.nn.sigmoid(
        jnp.dot(x11, wf_ref[...], preferred_element_type=jnp.float32) + bf_ref[...])
    o_ref[...] = jnp.dot(xc, wo_ref[...], preferred_element_type=jnp.float32) + bo_ref[...]


def _edge_meta(sorted_idx):
    tgt = (jnp.arange(NW + 1, dtype=jnp.int32) * NR)
    bounds = jnp.searchsorted(sorted_idx, tgt).astype(jnp.int32)
    elo = (bounds[:NW] // 8) * 8
    ehi = bounds[1:]
    meta = jnp.zeros((NW, 16), jnp.int32)
    meta = meta.at[:, 0].set(elo).at[:, 1].set(ehi)
    return meta.reshape(NW * 16)


def _edge_prep(edge_index):
    src = edge_index[0].astype(jnp.int32)
    dst = edge_index[1].astype(jnp.int32)
    perm = jnp.argsort(dst)
    dsts = dst[perm]
    srcs = src[perm]
    srcsort = jnp.sort(src)
    srcs_p = jnp.full((EP,), N, jnp.int32).at[:E].set(srcs)
    dsts_p = jnp.full((EP,), NPAD, jnp.int32).at[:E].set(dsts)
    srcsort_p = jnp.full((EP,), NPAD, jnp.int32).at[:E].set(srcsort)
    return srcs_p, dsts_p, srcsort_p, _edge_meta(dsts), _edge_meta(srcsort)


def _branch(x, srcs_p, dsts_p, meta_d, deg_in, deg_out, Wg, bg):
    dego_col = deg_out[:, None]
    degi_col = deg_in[:, None]

    xp = jnp.zeros((NPAD, H), jnp.float32).at[:N].set(x)
    y = _row_call(_scale_body, 2)(xp, dego_col)
    for i in range(NUM_NET):
        agg = _agg_call(y, srcs_p, dsts_p, meta_d)
        body = _dense_last_body if i == NUM_NET - 1 else _dense_mid_body
        y = _row_call(body, 5)(agg, dego_col, degi_col, Wg[i], bg[i][None, :])
    return y


def _meta_shift(m, p):
    m = m.reshape(NW, 16)
    return m.at[:, 0].add(p * EP).at[:, 1].add(p * EP).reshape(NW * 16)


def kernel(x0, edge_index0, x1, edge_index1, Wg1, bg1, Wg2, bg2,
           W_fc1, b_fc1, W_out, b_out):
    srcs1, dsts1, ssort1, md1, ms1 = _edge_prep(edge_index1)
    srcs0, dsts0, ssort0, md0, ms0 = _edge_prep(edge_index0)
    idx4 = jnp.concatenate([dsts1, ssort1, dsts0, ssort0])
    meta4 = jnp.concatenate([_meta_shift(md1, 0), _meta_shift(ms1, 1),
                             _meta_shift(md0, 2), _meta_shift(ms0, 3)])
    degs = _deg_call(idx4, meta4)[:, 0]
    h1 = _branch(x1, srcs1, dsts1, md1, degs[:NPAD], degs[NPAD:2 * NPAD],
                 Wg1, bg1)
    h0 = _branch(x0, srcs0, dsts0, md0, degs[2 * NPAD:3 * NPAD],
                 degs[3 * NPAD:], Wg2, bg2)
    m1, m0 = _pool_call(h1, h0)
    out = pl.pallas_call(
        _mlp_body,
        out_shape=jax.ShapeDtypeStruct((1, OUT), jnp.float32),
    )(m1, m0, W_fc1, b_fc1[None, :], W_out, b_out[None, :])
    return out[0]


# speedup vs baseline: 1.0175x; 1.0175x over previous
import functools

import jax
import jax.numpy as jnp
from jax import lax
from jax.experimental import pallas as pl
from jax.experimental.pallas import tpu as pltpu
from jax.experimental.pallas import tpu_sc as plsc

N = 10000
E = 160000
H = 256
OUT = 128
NUM_NET = 4

NC = 2
NS = 16
NW = NC * NS
NPAD = 10240
NR = NPAD // NW
NRT = NR + 8
CE = 64
SE = 2048
NCI = SE // CE
EP = E + SE + CE
ROWS = 512
NEG = -3.0e38

_mesh = plsc.VectorSubcoreMesh(
    core_axis_name="c", subcore_axis_name="s", num_cores=NC, num_subcores=NS)


def _agg_body(y_h, srcs_h, dsts_h, meta_h, out_h,
              src_blk, dst_blk, rows2, acc_b, meta_v, sem0, sem1):
    t = lax.axis_index("s") * NC + lax.axis_index("c")
    nr0 = t * NR
    zero16 = jnp.zeros((16,), jnp.float32)
    pltpu.sync_copy(meta_h, meta_v)
    mv = meta_v[pl.ds(t * 16, 16)]
    elo = mv[0]
    ehi = mv[1]

    def zrow(i, carry):
        for c in range(H // 16):
            acc_b[i, pl.ds(c * 16, 16)] = zero16
        return carry

    lax.fori_loop(0, NRT, zrow, 0)
    sems = (sem0, sem1)
    nsc = lax.div(ehi - elo + (SE - 1), jnp.int32(SE))

    def sc_loop(si, carry):
        sbase = pl.multiple_of(elo + si * SE, 8)
        pltpu.sync_copy(srcs_h.at[pl.ds(sbase, SE)], src_blk)
        pltpu.sync_copy(dsts_h.at[pl.ds(sbase, SE)], dst_blk)
        nin = jnp.minimum(jnp.int32(NCI),
                          lax.div(ehi - sbase + (CE - 1), jnp.int32(CE)))
        pltpu.async_copy(y_h.at[src_blk.at[pl.ds(0, CE)]], rows2.at[0], sem0)
        npair = lax.div(nin + 1, jnp.int32(2))

        def pair(j2, c2):
            for b in (0, 1):
                j = j2 * 2 + b

                @pl.when(j < nin)
                def _():
                    pltpu.make_async_copy(
                        y_h.at[src_blk.at[pl.ds(0, CE)]], rows2.at[b],
                        sems[b]).wait()

                    @pl.when(j + 1 < nin)
                    def _():
                        pltpu.async_copy(
                            y_h.at[src_blk.at[pl.ds((j + 1) * CE, CE)]],
                            rows2.at[1 - b], sems[1 - b])

                    def grp(g, c3):
                        dvr = dst_blk[pl.ds(j * CE + g * 16, 16)] - nr0
                        ok = (dvr >= 0) & (dvr < NR)
                        dv = jnp.where(ok, dvr, jnp.int32(NR))
                        for jj in range(16):
                            row = dv[jj]
                            e = g * 16 + jj
                            for c in range(H // 16):
                                plsc.addupdate(
                                    acc_b.at[row, pl.ds(c * 16, 16)],
                                    rows2[b, e, pl.ds(c * 16, 16)])
                        return c3

                    lax.fori_loop(0, CE // 16, grp, 0)
            return c2

        lax.fori_loop(0, npair, pair, 0)
        return carry

    lax.fori_loop(0, nsc, sc_loop, 0)
    pltpu.sync_copy(acc_b.at[pl.ds(0, NR)], out_h.at[pl.ds(nr0, NR)])


_agg_call = pl.kernel(
    _agg_body,
    out_type=jax.ShapeDtypeStruct((NPAD, H), jnp.float32),
    mesh=_mesh,
    scratch_types=[
        pltpu.VMEM((SE,), jnp.int32),
        pltpu.VMEM((SE,), jnp.int32),
        pltpu.VMEM((2, CE, H), jnp.float32),
        pltpu.VMEM((NRT, H), jnp.float32),
        pltpu.VMEM((NW * 16,), jnp.int32),
        pltpu.SemaphoreType.DMA,
        pltpu.SemaphoreType.DMA,
    ],
)


def _deg_body(idxs_h, meta_h, out_h, idx_blk, acc_b, meta_v):
    t = lax.axis_index("s") * NC + lax.axis_index("c")
    nr0 = t * NR
    zero16 = jnp.zeros((16,), jnp.float32)
    pltpu.sync_copy(meta_h, meta_v)
    mv = meta_v[pl.ds(t * 16, 16)]
    elo = mv[0]
    ehi = mv[1]

    def zrow(i, carry):
        acc_b[i, pl.ds(0, 16)] = zero16
        return carry

    lax.fori_loop(0, NRT, zrow, 0)
    onehot = jnp.where(lax.iota(jnp.int32, 16) == 0, 1.0, 0.0)
    nsc = lax.div(ehi - elo + (SE - 1), jnp.int32(SE))

    def sc_loop(si, carry):
        sbase = pl.multiple_of(elo + si * SE, 8)
        pltpu.sync_copy(idxs_h.at[pl.ds(sbase, SE)], idx_blk)
        ngr = jnp.minimum(jnp.int32(SE // 16),
                          lax.div(ehi - sbase + 15, jnp.int32(16)))

        def grp(g, c2):
            dvr = idx_blk[pl.ds(g * 16, 16)] - nr0
            ok = (dvr >= 0) & (dvr < NR)
            dv = jnp.where(ok, dvr, jnp.int32(NR))
            for jj in range(16):
                row = dv[jj]
                plsc.addupdate(acc_b.at[row, pl.ds(0, 16)], onehot)
            return c2

        lax.fori_loop(0, ngr, grp, 0)
        return carry

    lax.fori_loop(0, nsc, sc_loop, 0)
    pltpu.sync_copy(acc_b.at[pl.ds(0, NR)], out_h.at[pl.ds(nr0, NR)])


_deg_call = pl.kernel(
    _deg_body,
    out_type=jax.ShapeDtypeStruct((NPAD, 16), jnp.float32),
    mesh=_mesh,
    scratch_types=[
        pltpu.VMEM((SE,), jnp.int32),
        pltpu.VMEM((NRT, 16), jnp.float32),
        pltpu.VMEM((NW * 16,), jnp.int32),
    ],
)


def _scale_body(x_ref, dego_ref, o_ref):
    dego = dego_ref[...]
    ns = jnp.where(dego > 0, lax.rsqrt(dego), 0.0)
    o_ref[...] = x_ref[...] * ns


def _dense_mid_body(agg_ref, dego_ref, degi_ref, w_ref, b_ref, o_ref):
    dego = dego_ref[...]
    degi = degi_ref[...]
    ns = jnp.where(dego > 0, lax.rsqrt(dego), 0.0)
    nd = jnp.where(degi > 0, lax.rsqrt(degi), 0.0)
    acc = jnp.dot(agg_ref[...], w_ref[...], preferred_element_type=jnp.float32)
    o_ref[...] = acc * (ns * nd) + ns * b_ref[...]


def _dense_last_body(agg_ref, dego_ref, degi_ref, w_ref, b_ref, o_ref):
    degi = degi_ref[...]
    nd = jnp.where(degi > 0, lax.rsqrt(degi), 0.0)
    acc = jnp.dot(agg_ref[...], w_ref[...], preferred_element_type=jnp.float32)
    o_ref[...] = acc * nd + b_ref[...]


def _row_call(body, n_in):
    grid = (NPAD // ROWS,)
    specs = [pl.BlockSpec((ROWS, H), lambda i: (i, 0)),
             pl.BlockSpec((ROWS, 1), lambda i: (i, 0)),
             pl.BlockSpec((ROWS, 1), lambda i: (i, 0)),
             pl.BlockSpec((H, H), lambda i: (0, 0)),
             pl.BlockSpec((1, H), lambda i: (0, 0))][:n_in]
    return pl.pallas_call(
        body, grid=grid, in_specs=specs,
        out_specs=pl.BlockSpec((ROWS, H), lambda i: (i, 0)),
        out_shape=jax.ShapeDtypeStruct((NPAD, H), jnp.float32))


def _pool_body(h1_ref, h0_ref, o1_ref, o0_ref):
    i = pl.program_id(0)
    rows = lax.broadcasted_iota(jnp.int32, (ROWS, 1), 0) + i * ROWS
    valid = rows < N
    m1 = jnp.max(jnp.where(valid, h1_ref[...], NEG), axis=0, keepdims=True)
    m0 = jnp.max(jnp.where(valid, h0_ref[...], NEG), axis=0, keepdims=True)

    @pl.when(i == 0)
    def _():
        o1_ref[...] = jnp.full((1, H), NEG, jnp.float32)
        o0_ref[...] = jnp.full((1, H), NEG, jnp.float32)

    o1_ref[...] = jnp.maximum(o1_ref[...], m1)
    o0_ref[...] = jnp.maximum(o0_ref[...], m0)


_pool_call = pl.pallas_call(
    _pool_body, grid=(NPAD // ROWS,),
    in_specs=[pl.BlockSpec((ROWS, H), lambda i: (i, 0)),
              pl.BlockSpec((ROWS, H), lambda i: (i, 0))],
    out_specs=[pl.BlockSpec((1, H), lambda i: (0, 0)),
               pl.BlockSpec((1, H), lambda i: (0, 0))],
    out_shape=[jax.ShapeDtypeStruct((1, H), jnp.float32),
               jax.ShapeDtypeStruct((1, H), jnp.float32)])


def _mlp_body(m1_ref, m0_ref, wf_ref, bf_ref, wo_ref, bo_ref, o_ref):
    x11 = (m1_ref[...] + m0_ref[...]) * (1.0 / NUM_NET)
    xc = jax.nn.sigmoid(
        jnp.dot(x11, wf_ref[...], preferred_element_type=jnp.float32) + bf_ref[...])
    o_ref[...] = jnp.dot(xc, wo_ref[...], preferred_element_type=jnp.float32) + bo_ref[...]


def _edge_meta(sorted_idx):
    tgt = (jnp.arange(NW + 1, dtype=jnp.int32) * NR)
    bounds = jnp.searchsorted(sorted_idx, tgt).astype(jnp.int32)
    elo = (bounds[:NW] // 8) * 8
    ehi = bounds[1:]
    meta = jnp.zeros((NW, 16), jnp.int32)
    meta = meta.at[:, 0].set(elo).at[:, 1].set(ehi)
    return meta.reshape(NW * 16)


def _branch(x, edge_index, Wg, bg):
    src = edge_index[0].astype(jnp.int32)
    dst = edge_index[1].astype(jnp.int32)
    perm = jnp.argsort(dst)
    dsts = dst[perm]
    srcs = src[perm]
    srcsort = jnp.sort(src)
    srcs_p = jnp.full((EP,), N, jnp.int32).at[:E].set(srcs)
    dsts_p = jnp.full((EP,), NPAD, jnp.int32).at[:E].set(dsts)
    srcsort_p = jnp.full((EP,), NPAD, jnp.int32).at[:E].set(srcsort)
    meta_d = _edge_meta(dsts)
    meta_s = _edge_meta(srcsort)

    deg_in = _deg_call(dsts_p, meta_d)[:, 0]
    deg_out = _deg_call(srcsort_p, meta_s)[:, 0]
    dego_col = deg_out[:, None]
    degi_col = deg_in[:, None]

    xp = jnp.zeros((NPAD, H), jnp.float32).at[:N].set(x)
    y = _row_call(_scale_body, 2)(xp, dego_col)
    for i in range(NUM_NET):
        agg = _agg_call(y, srcs_p, dsts_p, meta_d)
        body = _dense_last_body if i == NUM_NET - 1 else _dense_mid_body
        y = _row_call(body, 5)(agg, dego_col, degi_col, Wg[i], bg[i][None, :])
    return y


def kernel(x0, edge_index0, x1, edge_index1, Wg1, bg1, Wg2, bg2,
           W_fc1, b_fc1, W_out, b_out):
    h1 = _branch(x1, edge_index1, Wg1, bg1)
    h0 = _branch(x0, edge_index0, Wg2, bg2)
    m1, m0 = _pool_call(h1, h0)
    out = pl.pallas_call(
        _mlp_body,
        out_shape=jax.ShapeDtypeStruct((1, OUT), jnp.float32),
    )(m1, m0, W_fc1, b_fc1[None, :], W_out, b_out[None, :])
    return out[0]
